# baseline (speedup 1.0000x reference)
import jax
import jax.numpy as jnp
from jax import lax
from jax.experimental import pallas as pl
from jax.experimental.pallas import tpu as pltpu

B, N, C, D, K = 4, 20000, 81, 256, 36
IOU_THRESH = 0.5
NP = 20480
ROWS, LANES = 160, 128
CHUNK = 8
NCHUNK = ROWS // CHUNK
NEG = -1e30


def _nms_body(s_ref, x1_ref, y1_ref, x2_ref, y2_ref,
              cl_hbm, feat_hbm,
              coords_out, feats_out, probs_out,
              s_scr, ar_scr, idx_smem, sem_f, sem_l):
    s_scr[...] = s_ref[...]
    ar_scr[...] = (x2_ref[...] - x1_ref[...]) * (y2_ref[...] - y1_ref[...])

    sub_iota = lax.broadcasted_iota(jnp.int32, (CHUNK, LANES), 0)
    lane_iota = lax.broadcasted_iota(jnp.int32, (CHUNK, LANES), 1)

    def initial_acc(b):
        macc = jnp.full((CHUNK, LANES), NEG, jnp.float32)
        iacc = jnp.zeros((CHUNK, LANES), jnp.int32)
        zc = jnp.zeros((CHUNK, LANES), jnp.float32)
        acc = [macc, iacc, zc, zc, zc, zc]
        for i in range(NCHUNK):
            sl = pl.ds(CHUNK * i, CHUNK)
            sc = s_scr[b, sl]
            upd = sc > acc[0]
            acc[0] = jnp.where(upd, sc, acc[0])
            acc[1] = jnp.where(upd, sub_iota + CHUNK * i, acc[1])
            acc[2] = jnp.where(upd, x1_ref[b, sl], acc[2])
            acc[3] = jnp.where(upd, y1_ref[b, sl], acc[3])
            acc[4] = jnp.where(upd, x2_ref[b, sl], acc[4])
            acc[5] = jnp.where(upd, y2_ref[b, sl], acc[5])
        return acc

    acc0 = []
    for b in range(B):
        acc0.extend(initial_acc(b))

    def round_body(k, carry):
        nxt = []
        for b in range(B):
            macc, iacc, cx1, cy1, cx2, cy2 = carry[6 * b:6 * b + 6]
            m = jnp.max(macc, axis=(0, 1), keepdims=True)
            flat = iacc * LANES + lane_iota
            idxv = jnp.min(jnp.where(macc == m, flat, jnp.int32(NP)),
                           axis=(0, 1), keepdims=True)
            idx_smem[b, k] = idxv[0, 0]
            win = jnp.logical_and(macc == m, flat == idxv)
            bx1 = jnp.max(jnp.where(win, cx1, NEG), axis=(0, 1), keepdims=True)
            by1 = jnp.max(jnp.where(win, cy1, NEG), axis=(0, 1), keepdims=True)
            bx2 = jnp.max(jnp.where(win, cx2, NEG), axis=(0, 1), keepdims=True)
            by2 = jnp.max(jnp.where(win, cy2, NEG), axis=(0, 1), keepdims=True)
            barea = (bx2 - bx1) * (by2 - by1)
            coords_out[b, pl.ds(k, 1), pl.ds(0, 1)] = bx1
            coords_out[b, pl.ds(k, 1), pl.ds(1, 1)] = by1
            coords_out[b, pl.ds(k, 1), pl.ds(2, 1)] = bx2
            coords_out[b, pl.ds(k, 1), pl.ds(3, 1)] = by2

            nm = jnp.full((CHUNK, LANES), NEG, jnp.float32)
            ni = jnp.zeros((CHUNK, LANES), jnp.int32)
            nx1 = jnp.zeros((CHUNK, LANES), jnp.float32)
            ny1 = nx1
            nx2 = nx1
            ny2 = nx1
            for i in range(NCHUNK):
                sl = pl.ds(CHUNK * i, CHUNK)
                x1 = x1_ref[b, sl]
                y1 = y1_ref[b, sl]
                x2 = x2_ref[b, sl]
                y2 = y2_ref[b, sl]
                ar = ar_scr[b, sl]
                xx1 = jnp.maximum(x1, bx1)
                yy1 = jnp.maximum(y1, by1)
                xx2 = jnp.minimum(x2, bx2)
                yy2 = jnp.minimum(y2, by2)
                inter = (jnp.maximum(xx2 - xx1, 0.0)
                         * jnp.maximum(yy2 - yy1, 0.0))
                iou = inter / (ar + barea - inter + 1e-9)
                snew = jnp.where(iou > IOU_THRESH, NEG, s_scr[b, sl])
                s_scr[b, sl] = snew
                upd = snew > nm
                nm = jnp.where(upd, snew, nm)
                ni = jnp.where(upd, sub_iota + CHUNK * i, ni)
                nx1 = jnp.where(upd, x1, nx1)
                ny1 = jnp.where(upd, y1, ny1)
                nx2 = jnp.where(upd, x2, nx2)
                ny2 = jnp.where(upd, y2, ny2)
            nxt.extend([nm, ni, nx1, ny1, nx2, ny2])
        return tuple(nxt)

    lax.fori_loop(0, K, round_body, tuple(acc0), unroll=False)

    copies = []
    for b in range(B):
        for k in range(K):
            i = idx_smem[b, k]
            i = jnp.minimum(jnp.maximum(i, 0), N - 1)
            fc = pltpu.make_async_copy(feat_hbm.at[b, i], feats_out.at[b, k], sem_f)
            lc = pltpu.make_async_copy(cl_hbm.at[b, i], probs_out.at[b, k], sem_l)
            fc.start()
            lc.start()
            copies.extend((fc, lc))
    for cp in copies:
        cp.wait()

    x = probs_out[...]
    mx = jnp.max(x, axis=-1, keepdims=True)
    e = jnp.exp(x - mx)
    probs_out[...] = e / jnp.sum(e, axis=-1, keepdims=True)


def kernel(boxes, scores, class_logits, features):
    pad = NP - N
    x1 = jnp.pad(boxes[:, :, 0], ((0, 0), (0, pad))).reshape(B, ROWS, LANES)
    y1 = jnp.pad(boxes[:, :, 1], ((0, 0), (0, pad))).reshape(B, ROWS, LANES)
    x2 = jnp.pad(boxes[:, :, 2], ((0, 0), (0, pad))).reshape(B, ROWS, LANES)
    y2 = jnp.pad(boxes[:, :, 3], ((0, 0), (0, pad))).reshape(B, ROWS, LANES)
    s = jnp.pad(scores, ((0, 0), (0, pad)), constant_values=NEG).reshape(B, ROWS, LANES)

    vmem = pl.BlockSpec(memory_space=pltpu.MemorySpace.VMEM)
    hbm = pl.BlockSpec(memory_space=pltpu.MemorySpace.HBM)
    coords, feats, probs = pl.pallas_call(
        _nms_body,
        in_specs=[vmem, vmem, vmem, vmem, vmem, hbm, hbm],
        out_specs=[vmem, vmem, vmem],
        out_shape=[
            jax.ShapeDtypeStruct((B, K, 4), jnp.float32),
            jax.ShapeDtypeStruct((B, K, D), jnp.float32),
            jax.ShapeDtypeStruct((B, K, C), jnp.float32),
        ],
        scratch_shapes=[
            pltpu.VMEM((B, ROWS, LANES), jnp.float32),
            pltpu.VMEM((B, ROWS, LANES), jnp.float32),
            pltpu.SMEM((B, K), jnp.int32),
            pltpu.SemaphoreType.DMA,
            pltpu.SemaphoreType.DMA,
        ],
    )(s, x1, y1, x2, y2, class_logits, features)
    return coords, feats, probs
